# Initial kernel scaffold; baseline (speedup 1.0000x reference)
import jax
import jax.numpy as jnp
from jax.experimental import pallas as pl


def kernel(inputs, table):
    raise NotImplementedError("write your pallas kernel here")


# speedup vs baseline: 3.3401x; 3.3401x over previous
import functools

import jax
import jax.numpy as jnp
from jax import lax
from jax.experimental import pallas as pl
from jax.experimental.pallas import tpu as pltpu, tpu_sc as plsc

VOCAB = 100000
EMB_DIM = 128
BATCH = 16384
N_FIELDS = 26
B_TOTAL = BATCH * N_FIELDS

_info = plsc.get_sparse_core_info()
_NC, _NS = _info.num_cores, _info.num_subcores
NW = _NC * _NS
B_PER_W = B_TOTAL // NW
CHUNK = 832
NCHUNK = B_PER_W // CHUNK

_mesh = plsc.VectorSubcoreMesh(core_axis_name="c", subcore_axis_name="s")


@functools.partial(
    pl.kernel,
    mesh=_mesh,
    out_type=jax.ShapeDtypeStruct((B_TOTAL, EMB_DIM), jnp.float32),
    scratch_types=[
        pltpu.VMEM((CHUNK,), jnp.int32),
        pltpu.VMEM((CHUNK, EMB_DIM), jnp.float32),
        pltpu.SemaphoreType.DMA,
    ],
)
def _sc_gather(idx_hbm, table_hbm, out_hbm, idx_v, rows_v, sem):
    wid = lax.axis_index("s") * _NC + lax.axis_index("c")
    base = wid * B_PER_W

    def body(g, carry):
        off = base + g * CHUNK
        pltpu.sync_copy(idx_hbm.at[pl.ds(off, CHUNK)], idx_v)
        pltpu.async_copy(table_hbm.at[idx_v], rows_v, sem).wait()
        pltpu.sync_copy(rows_v, out_hbm.at[pl.ds(off, CHUNK)])
        return carry

    lax.fori_loop(0, NCHUNK, body, 0)


def kernel(inputs, table):
    idx = inputs.reshape(-1).astype(jnp.int32)
    out = _sc_gather(idx, table)
    return out.reshape(inputs.shape + (EMB_DIM,))


# speedup vs baseline: 3.4000x; 1.0179x over previous
import functools

import jax
import jax.numpy as jnp
from jax import lax
from jax.experimental import pallas as pl
from jax.experimental.pallas import tpu as pltpu, tpu_sc as plsc

VOCAB = 100000
EMB_DIM = 128
BATCH = 16384
N_FIELDS = 26
B_TOTAL = BATCH * N_FIELDS

_info = plsc.get_sparse_core_info()
_NC, _NS = _info.num_cores, _info.num_subcores
NW = _NC * _NS
B_PER_W = B_TOTAL // NW
NBUF = 4
CHUNK = 208
NCHUNK = B_PER_W // CHUNK
NP = NCHUNK // NBUF

_mesh = plsc.VectorSubcoreMesh(core_axis_name="c", subcore_axis_name="s")


@functools.partial(
    pl.kernel,
    mesh=_mesh,
    out_type=jax.ShapeDtypeStruct((B_TOTAL, EMB_DIM), jnp.float32),
    scratch_types=(
        [pltpu.VMEM((CHUNK,), jnp.int32) for _ in range(NBUF)]
        + [pltpu.VMEM((CHUNK, EMB_DIM), jnp.float32) for _ in range(NBUF)]
        + [pltpu.SemaphoreType.DMA for _ in range(2 * NBUF)]
    ),
)
def _sc_gather(idx_hbm, table_hbm, out_hbm, *refs):
    idx_v = refs[0:NBUF]
    rows_v = refs[NBUF : 2 * NBUF]
    sg = refs[2 * NBUF : 3 * NBUF]
    so = refs[3 * NBUF : 4 * NBUF]

    wid = lax.axis_index("s") * _NC + lax.axis_index("c")
    base = wid * B_PER_W

    def fire_gather(g, b):
        pltpu.sync_copy(idx_hbm.at[pl.ds(base + g * CHUNK, CHUNK)], idx_v[b])
        pltpu.async_copy(table_hbm.at[idx_v[b]], rows_v[b], sg[b])

    def wait_gather(b):
        pltpu.make_async_copy(table_hbm.at[idx_v[b]], rows_v[b], sg[b]).wait()

    def fire_out(g, b):
        pltpu.async_copy(rows_v[b], out_hbm.at[pl.ds(base + g * CHUNK, CHUNK)], so[b])

    def wait_out(b):
        pltpu.make_async_copy(
            rows_v[b], out_hbm.at[pl.ds(base, CHUNK)], so[b]
        ).wait()

    fire_gather(0, 0)
    for b in range(1, NBUF):
        fire_gather(b, b)
        wait_gather(b - 1)
        fire_out(b - 1, b - 1)

    def body(p, carry):
        for b in range(NBUF):
            g = p * NBUF + b
            wait_out(b)
            fire_gather(g, b)
            b1 = (b - 1) % NBUF
            wait_gather(b1)
            fire_out(g - 1, b1)
        return carry

    lax.fori_loop(1, NP, body, 0)

    wait_gather(NBUF - 1)
    fire_out(NCHUNK - 1, NBUF - 1)
    for b in range(NBUF):
        wait_out(b)


def kernel(inputs, table):
    idx = inputs.reshape(-1).astype(jnp.int32)
    out = _sc_gather(idx, table)
    return out.reshape(inputs.shape + (EMB_DIM,))
